# baseline (device time: 95704 ns/iter reference)
import jax
import jax.numpy as jnp
from jax import lax
from jax.experimental import pallas as pl
from jax.experimental.pallas import tpu as pltpu


def kernel(x, dest):
    T, D = x.shape
    NBITS = T.bit_length()
    SUB = D // 128

    is_zero = dest == 0
    cz = jnp.cumsum(is_zero)
    c0s = cz[-1].astype(jnp.int32)
    pos = jnp.where(is_zero, cz - 1, c0s + jnp.cumsum(dest) - 1).astype(jnp.int32)
    c0 = c0s.reshape((1,))
    xb = x.astype(jnp.bfloat16).reshape(T, SUB, 128)
    x_sorted = (
        jnp.zeros((T, SUB, 128), jnp.bfloat16)
        .at[pos]
        .set(xb, unique_indices=True, mode="promise_in_bounds")
    )

    def body(x_ref, c0_ref, out_ref, send_sems, recv_sems, copy_sems):
        my_x = lax.axis_index("x")
        my_y = lax.axis_index("y")
        my_z = lax.axis_index("z")
        peer = (1 - my_x, my_y, my_z)

        barrier = pltpu.get_barrier_semaphore()
        pl.semaphore_signal(
            barrier, inc=1, device_id=peer, device_id_type=pl.DeviceIdType.MESH
        )
        pl.semaphore_wait(barrier, 1)

        c0v = c0_ref[0]
        zero = jnp.int32(0)
        is0 = my_x == 0
        own_off = jnp.where(is0, zero, c0v)
        own_len = jnp.where(is0, c0v, T - c0v)
        comm_len = T - own_len
        send_src = jnp.where(is0, c0v, zero)
        remote_dst = jnp.where(is0, zero, T - comm_len)

        soff = send_src
        roff = remote_dst
        for b in reversed(range(NBITS)):
            L = 1 << b
            take = comm_len & L

            @pl.when(take != 0)
            def _(soff=soff, roff=roff, L=L, b=b):
                pltpu.make_async_remote_copy(
                    src_ref=x_ref.at[pl.ds(soff, L)],
                    dst_ref=out_ref.at[pl.ds(roff, L)],
                    send_sem=send_sems.at[b],
                    recv_sem=recv_sems.at[b],
                    device_id=peer,
                    device_id_type=pl.DeviceIdType.MESH,
                ).start()

            soff = soff + take
            roff = roff + take

        off = own_off
        for b in reversed(range(NBITS)):
            L = 1 << b
            take = own_len & L

            @pl.when(take != 0)
            def _(off=off, L=L, b=b):
                pltpu.make_async_copy(
                    x_ref.at[pl.ds(off, L)],
                    out_ref.at[pl.ds(off, L)],
                    copy_sems.at[b],
                ).start()

            off = off + take

        for b in range(NBITS):
            L = 1 << b

            @pl.when((own_len & L) != 0)
            def _(L=L, b=b):
                pltpu.make_async_copy(
                    x_ref.at[pl.ds(0, L)],
                    out_ref.at[pl.ds(0, L)],
                    copy_sems.at[b],
                ).wait()

            @pl.when((comm_len & L) != 0)
            def _(L=L, b=b):
                rdma = pltpu.make_async_remote_copy(
                    src_ref=x_ref.at[pl.ds(0, L)],
                    dst_ref=out_ref.at[pl.ds(0, L)],
                    send_sem=send_sems.at[b],
                    recv_sem=recv_sems.at[b],
                    device_id=peer,
                    device_id_type=pl.DeviceIdType.MESH,
                )
                rdma.wait_send()
                rdma.wait_recv()

    out = pl.pallas_call(
        body,
        out_shape=jax.ShapeDtypeStruct((T, SUB, 128), jnp.bfloat16),
        in_specs=[
            pl.BlockSpec(memory_space=pl.ANY),
            pl.BlockSpec(memory_space=pltpu.SMEM),
        ],
        out_specs=pl.BlockSpec(memory_space=pl.ANY),
        scratch_shapes=[
            pltpu.SemaphoreType.DMA((NBITS,)),
            pltpu.SemaphoreType.DMA((NBITS,)),
            pltpu.SemaphoreType.DMA((NBITS,)),
        ],
        compiler_params=pltpu.CompilerParams(collective_id=0),
    )(x_sorted, c0)
    return out.reshape(T, D)


# device time: 52503 ns/iter; 1.8228x vs baseline; 1.8228x over previous
import jax
import jax.numpy as jnp
from jax import lax
from jax.experimental import pallas as pl
from jax.experimental.pallas import tpu as pltpu


def kernel(x, dest):
    T, D = x.shape
    NBITS = T.bit_length()
    SUB = D // 128

    combined = (dest * T).astype(jnp.int16) + jnp.arange(T, dtype=jnp.int16)
    order = jnp.sort(combined).astype(jnp.int32) & (T - 1)
    c0 = (T - jnp.sum(dest)).astype(jnp.int32).reshape((1,))
    x_sorted = jnp.take(x, order, axis=0).astype(jnp.bfloat16).reshape(T, SUB, 128)

    def body(x_ref, c0_ref, out_ref, send_sems, recv_sems, copy_sems):
        my_x = lax.axis_index("x")
        my_y = lax.axis_index("y")
        my_z = lax.axis_index("z")
        peer = (1 - my_x, my_y, my_z)

        barrier = pltpu.get_barrier_semaphore()
        pl.semaphore_signal(
            barrier, inc=1, device_id=peer, device_id_type=pl.DeviceIdType.MESH
        )
        pl.semaphore_wait(barrier, 1)

        c0v = c0_ref[0]
        zero = jnp.int32(0)
        is0 = my_x == 0
        own_off = jnp.where(is0, zero, c0v)
        own_len = jnp.where(is0, c0v, T - c0v)
        comm_len = T - own_len
        send_src = jnp.where(is0, c0v, zero)
        remote_dst = jnp.where(is0, zero, T - comm_len)

        soff = send_src
        roff = remote_dst
        for b in reversed(range(NBITS)):
            L = 1 << b
            take = comm_len & L

            @pl.when(take != 0)
            def _(soff=soff, roff=roff, L=L, b=b):
                pltpu.make_async_remote_copy(
                    src_ref=x_ref.at[pl.ds(soff, L)],
                    dst_ref=out_ref.at[pl.ds(roff, L)],
                    send_sem=send_sems.at[b],
                    recv_sem=recv_sems.at[b],
                    device_id=peer,
                    device_id_type=pl.DeviceIdType.MESH,
                ).start()

            soff = soff + take
            roff = roff + take

        off = own_off
        for b in reversed(range(NBITS)):
            L = 1 << b
            take = own_len & L

            @pl.when(take != 0)
            def _(off=off, L=L, b=b):
                pltpu.make_async_copy(
                    x_ref.at[pl.ds(off, L)],
                    out_ref.at[pl.ds(off, L)],
                    copy_sems.at[b],
                ).start()

            off = off + take

        for b in range(NBITS):
            L = 1 << b

            @pl.when((own_len & L) != 0)
            def _(L=L, b=b):
                pltpu.make_async_copy(
                    x_ref.at[pl.ds(0, L)],
                    out_ref.at[pl.ds(0, L)],
                    copy_sems.at[b],
                ).wait()

            @pl.when((comm_len & L) != 0)
            def _(L=L, b=b):
                rdma = pltpu.make_async_remote_copy(
                    src_ref=x_ref.at[pl.ds(0, L)],
                    dst_ref=out_ref.at[pl.ds(0, L)],
                    send_sem=send_sems.at[b],
                    recv_sem=recv_sems.at[b],
                    device_id=peer,
                    device_id_type=pl.DeviceIdType.MESH,
                )
                rdma.wait_send()
                rdma.wait_recv()

    out = pl.pallas_call(
        body,
        out_shape=jax.ShapeDtypeStruct((T, SUB, 128), jnp.bfloat16),
        in_specs=[
            pl.BlockSpec(memory_space=pl.ANY),
            pl.BlockSpec(memory_space=pltpu.SMEM),
        ],
        out_specs=pl.BlockSpec(memory_space=pl.ANY),
        scratch_shapes=[
            pltpu.SemaphoreType.DMA((NBITS,)),
            pltpu.SemaphoreType.DMA((NBITS,)),
            pltpu.SemaphoreType.DMA((NBITS,)),
        ],
        compiler_params=pltpu.CompilerParams(collective_id=0),
    )(x_sorted, c0)
    return out.reshape(T, D)


# device time: 51651 ns/iter; 1.8529x vs baseline; 1.0165x over previous
import jax
import jax.numpy as jnp
from jax import lax
from jax.experimental import pallas as pl
from jax.experimental.pallas import tpu as pltpu


def kernel(x, dest):
    T, D = x.shape
    NBITS = T.bit_length()
    SUB = D // 128

    combined = dest.astype(jnp.int32) * T + jnp.arange(T, dtype=jnp.int32)
    order = jnp.sort(combined) & (T - 1)
    c0 = (T - jnp.sum(dest)).astype(jnp.int32).reshape((1,))
    x_sorted = jnp.take(x, order, axis=0).astype(jnp.bfloat16).reshape(T, SUB, 128)

    def body(x_ref, c0_ref, out_ref, send_sems, recv_sems, copy_sems):
        my_x = lax.axis_index("x")
        my_y = lax.axis_index("y")
        my_z = lax.axis_index("z")
        peer = (1 - my_x, my_y, my_z)

        barrier = pltpu.get_barrier_semaphore()
        pl.semaphore_signal(
            barrier, inc=1, device_id=peer, device_id_type=pl.DeviceIdType.MESH
        )
        pl.semaphore_wait(barrier, 1)

        c0v = c0_ref[0]
        zero = jnp.int32(0)
        is0 = my_x == 0
        own_off = jnp.where(is0, zero, c0v)
        own_len = jnp.where(is0, c0v, T - c0v)
        comm_len = T - own_len
        send_src = jnp.where(is0, c0v, zero)
        remote_dst = jnp.where(is0, zero, T - comm_len)

        soff = send_src
        roff = remote_dst
        for b in reversed(range(NBITS)):
            L = 1 << b
            take = comm_len & L

            @pl.when(take != 0)
            def _(soff=soff, roff=roff, L=L, b=b):
                pltpu.make_async_remote_copy(
                    src_ref=x_ref.at[pl.ds(soff, L)],
                    dst_ref=out_ref.at[pl.ds(roff, L)],
                    send_sem=send_sems.at[b],
                    recv_sem=recv_sems.at[b],
                    device_id=peer,
                    device_id_type=pl.DeviceIdType.MESH,
                ).start()

            soff = soff + take
            roff = roff + take

        off = own_off
        for b in reversed(range(NBITS)):
            L = 1 << b
            take = own_len & L

            @pl.when(take != 0)
            def _(off=off, L=L, b=b):
                pltpu.make_async_copy(
                    x_ref.at[pl.ds(off, L)],
                    out_ref.at[pl.ds(off, L)],
                    copy_sems.at[b],
                ).start()

            off = off + take

        for b in range(NBITS):
            L = 1 << b

            @pl.when((own_len & L) != 0)
            def _(L=L, b=b):
                pltpu.make_async_copy(
                    x_ref.at[pl.ds(0, L)],
                    out_ref.at[pl.ds(0, L)],
                    copy_sems.at[b],
                ).wait()

            @pl.when((comm_len & L) != 0)
            def _(L=L, b=b):
                rdma = pltpu.make_async_remote_copy(
                    src_ref=x_ref.at[pl.ds(0, L)],
                    dst_ref=out_ref.at[pl.ds(0, L)],
                    send_sem=send_sems.at[b],
                    recv_sem=recv_sems.at[b],
                    device_id=peer,
                    device_id_type=pl.DeviceIdType.MESH,
                )
                rdma.wait_send()
                rdma.wait_recv()

    out = pl.pallas_call(
        body,
        out_shape=jax.ShapeDtypeStruct((T, SUB, 128), jnp.bfloat16),
        in_specs=[
            pl.BlockSpec(memory_space=pl.ANY),
            pl.BlockSpec(memory_space=pltpu.SMEM),
        ],
        out_specs=pl.BlockSpec(memory_space=pl.ANY),
        scratch_shapes=[
            pltpu.SemaphoreType.DMA((NBITS,)),
            pltpu.SemaphoreType.DMA((NBITS,)),
            pltpu.SemaphoreType.DMA((NBITS,)),
        ],
        compiler_params=pltpu.CompilerParams(collective_id=0),
    )(x_sorted, c0)
    return out.reshape(T, D)


# device time: 44116.746 ns/iter; 2.1693x vs baseline; 1.1708x over previous
import jax
import jax.numpy as jnp
from jax import lax
from jax.experimental import pallas as pl
from jax.experimental.pallas import tpu as pltpu


def kernel(x, dest):
    T, D = x.shape
    NBITS = T.bit_length()
    SUB = D // 128

    combined = dest.astype(jnp.int32) * T + jnp.arange(T, dtype=jnp.int32)
    order = jnp.sort(combined) & (T - 1)
    c0 = (T - jnp.sum(dest)).astype(jnp.int32).reshape((1,))
    x_sorted = (
        x.at[order]
        .get(unique_indices=True, mode="promise_in_bounds")
        .astype(jnp.bfloat16)
        .reshape(T, SUB, 128)
    )

    def body(x_ref, c0_ref, out_ref, send_sems, recv_sems, copy_sems):
        my_x = lax.axis_index("x")
        my_y = lax.axis_index("y")
        my_z = lax.axis_index("z")
        peer = (1 - my_x, my_y, my_z)

        barrier = pltpu.get_barrier_semaphore()
        pl.semaphore_signal(
            barrier, inc=1, device_id=peer, device_id_type=pl.DeviceIdType.MESH
        )
        pl.semaphore_wait(barrier, 1)

        c0v = c0_ref[0]
        zero = jnp.int32(0)
        is0 = my_x == 0
        own_off = jnp.where(is0, zero, c0v)
        own_len = jnp.where(is0, c0v, T - c0v)
        comm_len = T - own_len
        send_src = jnp.where(is0, c0v, zero)
        remote_dst = jnp.where(is0, zero, T - comm_len)

        soff = send_src
        roff = remote_dst
        for b in reversed(range(NBITS)):
            L = 1 << b
            take = comm_len & L

            @pl.when(take != 0)
            def _(soff=soff, roff=roff, L=L, b=b):
                pltpu.make_async_remote_copy(
                    src_ref=x_ref.at[pl.ds(soff, L)],
                    dst_ref=out_ref.at[pl.ds(roff, L)],
                    send_sem=send_sems.at[b],
                    recv_sem=recv_sems.at[b],
                    device_id=peer,
                    device_id_type=pl.DeviceIdType.MESH,
                ).start()

            soff = soff + take
            roff = roff + take

        off = own_off
        for b in reversed(range(NBITS)):
            L = 1 << b
            take = own_len & L

            @pl.when(take != 0)
            def _(off=off, L=L, b=b):
                pltpu.make_async_copy(
                    x_ref.at[pl.ds(off, L)],
                    out_ref.at[pl.ds(off, L)],
                    copy_sems.at[b],
                ).start()

            off = off + take

        for b in range(NBITS):
            L = 1 << b

            @pl.when((own_len & L) != 0)
            def _(L=L, b=b):
                pltpu.make_async_copy(
                    x_ref.at[pl.ds(0, L)],
                    out_ref.at[pl.ds(0, L)],
                    copy_sems.at[b],
                ).wait()

            @pl.when((comm_len & L) != 0)
            def _(L=L, b=b):
                rdma = pltpu.make_async_remote_copy(
                    src_ref=x_ref.at[pl.ds(0, L)],
                    dst_ref=out_ref.at[pl.ds(0, L)],
                    send_sem=send_sems.at[b],
                    recv_sem=recv_sems.at[b],
                    device_id=peer,
                    device_id_type=pl.DeviceIdType.MESH,
                )
                rdma.wait_send()
                rdma.wait_recv()

    out = pl.pallas_call(
        body,
        out_shape=jax.ShapeDtypeStruct((T, SUB, 128), jnp.bfloat16),
        in_specs=[
            pl.BlockSpec(memory_space=pl.ANY),
            pl.BlockSpec(memory_space=pltpu.SMEM),
        ],
        out_specs=pl.BlockSpec(memory_space=pl.ANY),
        scratch_shapes=[
            pltpu.SemaphoreType.DMA((NBITS,)),
            pltpu.SemaphoreType.DMA((NBITS,)),
            pltpu.SemaphoreType.DMA((NBITS,)),
        ],
        compiler_params=pltpu.CompilerParams(collective_id=0),
    )(x_sorted, c0)
    return out.reshape(T, D)
